# baseline (speedup 1.0000x reference)
import functools

import jax
import jax.numpy as jnp
from jax import lax
from jax.experimental import pallas as pl
from jax.experimental.pallas import tpu as pltpu

E = 8
TOPK = 2
D = 1024
DFF = 4096
M = 512
NT = 15
NROWS = NT * M
F_BLK = 1024
K1 = DFF // F_BLK

_INTERPRET = False


def _gmm1_body(te_ref, x_ref, wg_ref, wu_ref, ws_ref, act_ref,
               wg_scr, wu_scr):
    t = pl.program_id(1)
    prev = te_ref[jnp.maximum(t - 1, 0)]
    changed = jnp.logical_or(t == 0, te_ref[t] != prev)

    @pl.when(changed)
    def _cast():
        wg_scr[...] = wg_ref[0].astype(jnp.bfloat16)
        wu_scr[...] = wu_ref[0].astype(jnp.bfloat16)

    x = x_ref[pl.ds(t * M, M), :]
    g = lax.dot_general(x, wg_scr[...], (((1,), (1,)), ((), ())),
                        preferred_element_type=jnp.float32)
    u = lax.dot_general(x, wu_scr[...], (((1,), (1,)), ((), ())),
                        preferred_element_type=jnp.float32)
    act = (g * jax.nn.sigmoid(g)) * u
    w = ws_ref[0, 0, :]
    act_ref[...] = (act * w[:, None]).astype(jnp.bfloat16)


def _gmm2_body(te_ref, act_ref, wd_ref, out_ref, wd_scr):
    t = pl.program_id(0)
    prev = te_ref[jnp.maximum(t - 1, 0)]
    changed = jnp.logical_or(t == 0, te_ref[t] != prev)

    @pl.when(changed)
    def _cast():
        wd_scr[...] = wd_ref[0].astype(jnp.bfloat16)

    out_ref[...] = lax.dot_general(act_ref[...], wd_scr[...],
                                   (((1,), (1,)), ((), ())),
                                   preferred_element_type=jnp.float32)


def kernel(hidden_states, residual, ln_weight, ln_bias, router_weight,
           gate_up_proj, down_proj):
    B, S, _ = hidden_states.shape
    T = B * S

    h3 = residual + hidden_states
    mu = jnp.mean(h3, axis=-1, keepdims=True)
    var = jnp.mean((h3 - mu) ** 2, axis=-1, keepdims=True)
    normed = (h3 - mu) * lax.rsqrt(var + 1e-5) * ln_weight + ln_bias
    hidden_2d = normed.reshape(T, D)
    logits = hidden_2d @ router_weight.T
    probs = jax.nn.softmax(logits, axis=-1)
    routing_weights, selected_experts = lax.top_k(probs, TOPK)
    routing_weights = routing_weights / jnp.sum(routing_weights, axis=-1,
                                                keepdims=True)

    eflat = selected_experts.reshape(-1).astype(jnp.int32)
    rwflat = routing_weights.reshape(-1)
    oh = jax.nn.one_hot(eflat, E, dtype=jnp.int32)
    csum = jnp.cumsum(oh, axis=0)
    counts = csum[-1]
    rank = jnp.take_along_axis(csum, eflat[:, None], axis=1)[:, 0] - 1
    ntiles = (counts + M - 1) // M
    tile_end = jnp.cumsum(ntiles)
    tile_start = tile_end - ntiles
    row_start = tile_start * M
    pos = jnp.take(row_start, eflat) + rank
    tile_ids = jnp.arange(NT, dtype=jnp.int32)
    te = jnp.searchsorted(tile_end, tile_ids, side='right').astype(jnp.int32)
    te = jnp.minimum(te, E - 1)
    gidx = jnp.zeros((NROWS,), jnp.int32).at[pos].set(
        jnp.arange(2 * T, dtype=jnp.int32) // TOPK)
    ws = jnp.zeros((NROWS,), jnp.float32).at[pos].set(rwflat)

    xbf = hidden_2d.astype(jnp.bfloat16)
    x_sorted = jnp.take(xbf, gidx, axis=0)

    ws3 = ws.reshape(NT, 1, M)
    act = pl.pallas_call(
        _gmm1_body,
        grid_spec=pltpu.PrefetchScalarGridSpec(
            num_scalar_prefetch=1,
            grid=(K1, NT),
            in_specs=[
                pl.BlockSpec((NROWS, D), lambda k, t, te: (0, 0)),
                pl.BlockSpec((1, F_BLK, D), lambda k, t, te: (te[t], k, 0)),
                pl.BlockSpec((1, F_BLK, D),
                             lambda k, t, te: (te[t], K1 + k, 0)),
                pl.BlockSpec((1, 1, M), lambda k, t, te: (t, 0, 0)),
            ],
            out_specs=pl.BlockSpec((M, F_BLK), lambda k, t, te: (t, k)),
            scratch_shapes=[
                pltpu.VMEM((F_BLK, D), jnp.bfloat16),
                pltpu.VMEM((F_BLK, D), jnp.bfloat16),
            ],
        ),
        out_shape=jax.ShapeDtypeStruct((NROWS, DFF), jnp.bfloat16),
        interpret=_INTERPRET,
    )(te, x_sorted, gate_up_proj, gate_up_proj, ws3)

    y_sorted = pl.pallas_call(
        _gmm2_body,
        grid_spec=pltpu.PrefetchScalarGridSpec(
            num_scalar_prefetch=1,
            grid=(NT,),
            in_specs=[
                pl.BlockSpec((M, DFF), lambda t, te: (t, 0)),
                pl.BlockSpec((1, D, DFF), lambda t, te: (te[t], 0, 0)),
            ],
            out_specs=pl.BlockSpec((M, D), lambda t, te: (t, 0)),
            scratch_shapes=[
                pltpu.VMEM((D, DFF), jnp.bfloat16),
            ],
        ),
        out_shape=jax.ShapeDtypeStruct((NROWS, D), jnp.float32),
        interpret=_INTERPRET,
    )(te, act, down_proj)

    pos2 = pos.reshape(T, TOPK)
    y1 = jnp.take(y_sorted, pos2[:, 0], axis=0)
    y2 = jnp.take(y_sorted, pos2[:, 1], axis=0)
    out = h3.reshape(T, D) + y1 + y2
    return out.reshape(B, S, D)
